# initial kernel scaffold (unmeasured)
import math

import jax
import jax.numpy as jnp
from jax import lax
from jax.experimental import pallas as pl
from jax.experimental.pallas import tpu as pltpu

N_DEV = 4
QC = 512


def kernel(q, k, v):
    S, D = q.shape
    n_chunks = S // QC
    scale = 1.0 / math.sqrt(D)

    def body(q_ref, k_ref, v_ref, out_ref, kcomm, vcomm,
             ksend, krecv, vsend, vrecv):
        my = lax.axis_index("i")
        right = lax.rem(my + 1, N_DEV)
        left = lax.rem(my + N_DEV - 1, N_DEV)

        barrier = pltpu.get_barrier_semaphore()
        for nbr in (left, right):
            pl.semaphore_signal(barrier, inc=1, device_id=(nbr,),
                                device_id_type=pl.DeviceIdType.MESH)
        pl.semaphore_wait(barrier, 2)

        rdmas = []
        m = [None] * n_chunks
        l = [None] * n_chunks
        acc = [None] * n_chunks

        for s in range(N_DEV):
            if s > 0:
                for r in rdmas[s - 1]:
                    r.wait()
            if s < N_DEV - 1:
                ksrc = k_ref if s == 0 else kcomm.at[s - 1]
                vsrc = v_ref if s == 0 else vcomm.at[s - 1]
                kr = pltpu.make_async_remote_copy(
                    src_ref=ksrc, dst_ref=kcomm.at[s],
                    send_sem=ksend.at[s], recv_sem=krecv.at[s],
                    device_id=(right,), device_id_type=pl.DeviceIdType.MESH)
                vr = pltpu.make_async_remote_copy(
                    src_ref=vsrc, dst_ref=vcomm.at[s],
                    send_sem=vsend.at[s], recv_sem=vrecv.at[s],
                    device_id=(right,), device_id_type=pl.DeviceIdType.MESH)
                kr.start()
                vr.start()
                rdmas.append((kr, vr))

            k_cur = k_ref[...] if s == 0 else kcomm[s - 1]
            v_cur = v_ref[...] if s == 0 else vcomm[s - 1]
            for c in range(n_chunks):
                qc = q_ref[pl.ds(c * QC, QC), :]
                scores = lax.dot_general(
                    qc, k_cur, (((1,), (1,)), ((), ())),
                    preferred_element_type=jnp.float32) * scale
                if s == 0:
                    m[c] = jnp.max(scores, axis=1, keepdims=True)
                    p = jnp.exp(scores - m[c])
                    l[c] = jnp.sum(p, axis=1, keepdims=True)
                    acc[c] = lax.dot_general(
                        p, v_cur, (((1,), (0,)), ((), ())),
                        preferred_element_type=jnp.float32)
                else:
                    m_new = jnp.maximum(
                        m[c], jnp.max(scores, axis=1, keepdims=True))
                    p = jnp.exp(scores - m_new)
                    corr = jnp.exp(m[c] - m_new)
                    l[c] = l[c] * corr + jnp.sum(p, axis=1, keepdims=True)
                    acc[c] = acc[c] * corr + lax.dot_general(
                        p, v_cur, (((1,), (0,)), ((), ())),
                        preferred_element_type=jnp.float32)
                    m[c] = m_new

        for c in range(n_chunks):
            out_ref[pl.ds(c * QC, QC), :] = acc[c] / l[c]

    return pl.pallas_call(
        body,
        out_shape=jax.ShapeDtypeStruct((S, D), jnp.float32),
        in_specs=[pl.BlockSpec(memory_space=pltpu.VMEM)] * 3,
        out_specs=pl.BlockSpec(memory_space=pltpu.VMEM),
        scratch_shapes=[
            pltpu.VMEM((N_DEV - 1, S, D), jnp.float32),
            pltpu.VMEM((N_DEV - 1, S, D), jnp.float32),
            pltpu.SemaphoreType.DMA((N_DEV - 1,)),
            pltpu.SemaphoreType.DMA((N_DEV - 1,)),
            pltpu.SemaphoreType.DMA((N_DEV - 1,)),
            pltpu.SemaphoreType.DMA((N_DEV - 1,)),
        ],
        compiler_params=pltpu.CompilerParams(collective_id=0),
    )(q, k, v)


# baseline (device time: 702529 ns/iter reference)
import math

import jax
import jax.numpy as jnp
from jax import lax
from jax.experimental import pallas as pl
from jax.experimental.pallas import tpu as pltpu

N_DEV = 4
QC = 256
KC = 512


def kernel(q, k, v):
    S, D = q.shape
    n_chunks = S // QC
    scale = 1.0 / math.sqrt(D)

    def body(q_ref, k_ref, v_ref, out_ref, kbuf, vbuf, m_scr, l_scr,
             copy_sems, ksend, krecv, vsend, vrecv, credit):
        my = lax.axis_index("i")
        right = lax.rem(my + 1, N_DEV)
        left = lax.rem(my + N_DEV - 1, N_DEV)

        ck = pltpu.make_async_copy(k_ref, kbuf.at[0], copy_sems.at[0])
        cv = pltpu.make_async_copy(v_ref, vbuf.at[0], copy_sems.at[1])
        ck.start()
        cv.start()
        ck.wait()
        cv.wait()

        barrier = pltpu.get_barrier_semaphore()
        for nbr in (left, right):
            pl.semaphore_signal(barrier, inc=1, device_id=(nbr,),
                                device_id_type=pl.DeviceIdType.MESH)
        pl.semaphore_wait(barrier, 2)

        for s in range(N_DEV):
            src_slot = s % 2
            dst_slot = (s + 1) % 2

            if s < N_DEV - 1:
                if s >= 1:
                    pl.semaphore_wait(credit, 1)
                kr = pltpu.make_async_remote_copy(
                    src_ref=kbuf.at[src_slot], dst_ref=kbuf.at[dst_slot],
                    send_sem=ksend.at[s], recv_sem=krecv.at[s],
                    device_id=(right,), device_id_type=pl.DeviceIdType.MESH)
                vr = pltpu.make_async_remote_copy(
                    src_ref=vbuf.at[src_slot], dst_ref=vbuf.at[dst_slot],
                    send_sem=vsend.at[s], recv_sem=vrecv.at[s],
                    device_id=(right,), device_id_type=pl.DeviceIdType.MESH)
                kr.start()
                vr.start()

            first = s == 0
            last = s == N_DEV - 1
            n_kv = S // KC

            def chunk_step(c, _, src_slot=src_slot, first=first, last=last):
                rows = pl.ds(c * QC, QC)
                qc = q_ref[rows, :]

                def kv_step(j, carry, qc=qc):
                    m_old, l_old, acc = carry
                    kv_rows = pl.ds(j * KC, KC)
                    kj = kbuf[src_slot, kv_rows, :]
                    vj = vbuf[src_slot, kv_rows, :]
                    sc = lax.dot_general(
                        qc, kj, (((1,), (1,)), ((), ())),
                        preferred_element_type=jnp.float32) * scale
                    m_new = jnp.maximum(
                        m_old, jnp.max(sc, axis=1, keepdims=True))
                    p = jnp.exp(sc - m_new)
                    corr = jnp.exp(m_old - m_new)
                    l_new = l_old * corr + jnp.sum(p, axis=1, keepdims=True)
                    acc = acc * corr + lax.dot_general(
                        p, vj, (((1,), (0,)), ((), ())),
                        preferred_element_type=jnp.float32)
                    return m_new, l_new, acc

                if first:
                    init = (jnp.full((QC, 1), -1e30, jnp.float32),
                            jnp.zeros((QC, 1), jnp.float32),
                            jnp.zeros((QC, D), jnp.float32))
                else:
                    init = (m_scr[rows, :], l_scr[rows, :], out_ref[rows, :])
                m_new, l_new, acc = lax.fori_loop(0, n_kv, kv_step, init)
                if last:
                    out_ref[rows, :] = acc / l_new
                else:
                    out_ref[rows, :] = acc
                    m_scr[rows, :] = m_new
                    l_scr[rows, :] = l_new
                return 0

            lax.fori_loop(0, n_chunks, chunk_step, 0)

            if s < N_DEV - 1:
                kr.wait_send()
                vr.wait_send()
                if s < N_DEV - 2:
                    pl.semaphore_signal(
                        credit, inc=1, device_id=(left,),
                        device_id_type=pl.DeviceIdType.MESH)
                kr.wait_recv()
                vr.wait_recv()

    return pl.pallas_call(
        body,
        out_shape=jax.ShapeDtypeStruct((S, D), jnp.float32),
        in_specs=[
            pl.BlockSpec(memory_space=pltpu.VMEM),
            pl.BlockSpec(memory_space=pl.ANY),
            pl.BlockSpec(memory_space=pl.ANY),
        ],
        out_specs=pl.BlockSpec(memory_space=pltpu.VMEM),
        scratch_shapes=[
            pltpu.VMEM((2, S, D), jnp.float32),
            pltpu.VMEM((2, S, D), jnp.float32),
            pltpu.VMEM((S, 1), jnp.float32),
            pltpu.VMEM((S, 1), jnp.float32),
            pltpu.SemaphoreType.DMA((2,)),
            pltpu.SemaphoreType.DMA((N_DEV - 1,)),
            pltpu.SemaphoreType.DMA((N_DEV - 1,)),
            pltpu.SemaphoreType.DMA((N_DEV - 1,)),
            pltpu.SemaphoreType.DMA((N_DEV - 1,)),
            pltpu.SemaphoreType.REGULAR,
        ],
        compiler_params=pltpu.CompilerParams(
            collective_id=0,
            vmem_limit_bytes=64 * 1024 * 1024,
        ),
    )(q, k, v)


# device time: 555446 ns/iter; 1.2648x vs baseline; 1.2648x over previous
import math

import jax
import jax.numpy as jnp
from jax import lax
from jax.experimental import pallas as pl
from jax.experimental.pallas import tpu as pltpu

N_DEV = 4
QC = 256
KC = 512
CC = 512


def kernel(q, k, v):
    S, D = q.shape
    n_chunks = S // QC
    scale = 1.0 / math.sqrt(D)

    def body(q_ref, k_ref, v_ref, out_ref, kbuf, vbuf, m_scr, l_scr,
             ksend, krecv, vsend, vrecv, credit):
        my = lax.axis_index("i")
        right = lax.rem(my + 1, N_DEV)
        left = lax.rem(my + N_DEV - 1, N_DEV)

        def cast_step(c, _):
            rows = pl.ds(c * CC, CC)
            kbuf[0, rows, :] = k_ref[rows, :].astype(jnp.bfloat16)
            vbuf[0, rows, :] = v_ref[rows, :].astype(jnp.bfloat16)
            return 0

        lax.fori_loop(0, S // CC, cast_step, 0)

        barrier = pltpu.get_barrier_semaphore()
        for nbr in (left, right):
            pl.semaphore_signal(barrier, inc=1, device_id=(nbr,),
                                device_id_type=pl.DeviceIdType.MESH)
        pl.semaphore_wait(barrier, 2)

        for s in range(N_DEV):
            src_slot = s % 2
            dst_slot = (s + 1) % 2

            if s < N_DEV - 1:
                if s >= 1:
                    pl.semaphore_wait(credit, 1)
                kr = pltpu.make_async_remote_copy(
                    src_ref=kbuf.at[src_slot], dst_ref=kbuf.at[dst_slot],
                    send_sem=ksend.at[s], recv_sem=krecv.at[s],
                    device_id=(right,), device_id_type=pl.DeviceIdType.MESH)
                vr = pltpu.make_async_remote_copy(
                    src_ref=vbuf.at[src_slot], dst_ref=vbuf.at[dst_slot],
                    send_sem=vsend.at[s], recv_sem=vrecv.at[s],
                    device_id=(right,), device_id_type=pl.DeviceIdType.MESH)
                kr.start()
                vr.start()

            first = s == 0
            last = s == N_DEV - 1
            n_kv = S // KC

            def chunk_step(c, _, src_slot=src_slot, first=first, last=last):
                rows = pl.ds(c * QC, QC)
                qc = q_ref[rows, :].astype(jnp.bfloat16)

                def kv_step(j, carry, qc=qc):
                    m_old, l_old, acc = carry
                    kv_rows = pl.ds(j * KC, KC)
                    kj = kbuf[src_slot, kv_rows, :]
                    vj = vbuf[src_slot, kv_rows, :]
                    sc = lax.dot_general(
                        qc, kj, (((1,), (1,)), ((), ())),
                        preferred_element_type=jnp.float32) * scale
                    m_new = jnp.maximum(
                        m_old, jnp.max(sc, axis=1, keepdims=True))
                    p = jnp.exp(sc - m_new)
                    corr = jnp.exp(m_old - m_new)
                    l_new = l_old * corr + jnp.sum(p, axis=1, keepdims=True)
                    acc = acc * corr + lax.dot_general(
                        p.astype(jnp.bfloat16), vj, (((1,), (0,)), ((), ())),
                        preferred_element_type=jnp.float32)
                    return m_new, l_new, acc

                if first:
                    init = (jnp.full((QC, 1), -1e30, jnp.float32),
                            jnp.zeros((QC, 1), jnp.float32),
                            jnp.zeros((QC, D), jnp.float32))
                else:
                    init = (m_scr[rows, :], l_scr[rows, :], out_ref[rows, :])
                m_new, l_new, acc = lax.fori_loop(0, n_kv, kv_step, init)
                if last:
                    out_ref[rows, :] = acc / l_new
                else:
                    out_ref[rows, :] = acc
                    m_scr[rows, :] = m_new
                    l_scr[rows, :] = l_new
                return 0

            lax.fori_loop(0, n_chunks, chunk_step, 0)

            if s < N_DEV - 1:
                kr.wait_send()
                vr.wait_send()
                if s < N_DEV - 2:
                    pl.semaphore_signal(
                        credit, inc=1, device_id=(left,),
                        device_id_type=pl.DeviceIdType.MESH)
                kr.wait_recv()
                vr.wait_recv()

    return pl.pallas_call(
        body,
        out_shape=jax.ShapeDtypeStruct((S, D), jnp.float32),
        in_specs=[
            pl.BlockSpec(memory_space=pltpu.VMEM),
            pl.BlockSpec(memory_space=pltpu.VMEM),
            pl.BlockSpec(memory_space=pltpu.VMEM),
        ],
        out_specs=pl.BlockSpec(memory_space=pltpu.VMEM),
        scratch_shapes=[
            pltpu.VMEM((2, S, D), jnp.bfloat16),
            pltpu.VMEM((2, S, D), jnp.bfloat16),
            pltpu.VMEM((S, 1), jnp.float32),
            pltpu.VMEM((S, 1), jnp.float32),
            pltpu.SemaphoreType.DMA((N_DEV - 1,)),
            pltpu.SemaphoreType.DMA((N_DEV - 1,)),
            pltpu.SemaphoreType.DMA((N_DEV - 1,)),
            pltpu.SemaphoreType.DMA((N_DEV - 1,)),
            pltpu.SemaphoreType.REGULAR,
        ],
        compiler_params=pltpu.CompilerParams(
            collective_id=0,
            vmem_limit_bytes=64 * 1024 * 1024,
        ),
    )(q, k, v)
